# baseline (speedup 1.0000x reference)
import functools

import jax
import jax.numpy as jnp
from jax.experimental import pallas as pl
from jax.experimental.pallas import tpu as pltpu

_BN_EPS = 1e-5
_VMEM_LIMIT = 58 * 1024 * 1024


def _fused_kernel(x_ref, w_ref, b_ref, o_ref, *, inv_n):
    lin = jnp.dot(x_ref[...], w_ref[...],
                  preferred_element_type=jnp.float32) + b_ref[...]
    y = jnp.tanh(lin)
    s = jnp.sum(y)
    ss = jnp.sum(y * y)
    mean = s * inv_n
    var = jnp.maximum(ss * inv_n - mean * mean, 0.0)
    inv_std = jax.lax.rsqrt(var + jnp.float32(_BN_EPS))
    o_ref[...] = (y - mean) * inv_std


def kernel(x, weight, bias):
    B, G, F = x.shape
    GF = G * F

    x = x.astype(jnp.float32)
    weight = weight.astype(jnp.float32)
    bias = bias.astype(jnp.float32).reshape(1, G)

    x_flat = x.reshape(B, GF)

    w_bd = (weight[:, :, None] * jnp.eye(G, dtype=jnp.float32)[:, None, :]
            ).reshape(GF, G)

    fk = functools.partial(_fused_kernel, inv_n=1.0 / float(B * G))
    out = pl.pallas_call(
        fk,
        out_shape=jax.ShapeDtypeStruct((B, G), jnp.float32),
        in_specs=[
            pl.BlockSpec((B, GF), lambda: (0, 0)),
            pl.BlockSpec((GF, G), lambda: (0, 0)),
            pl.BlockSpec((1, G), lambda: (0, 0)),
        ],
        out_specs=pl.BlockSpec((B, G), lambda: (0, 0)),
        compiler_params=pltpu.CompilerParams(
            vmem_limit_bytes=_VMEM_LIMIT,
        ),
    )(x_flat, w_bd, bias)

    return out.reshape(B, G, 1)


# speedup vs baseline: 1.0701x; 1.0701x over previous
import functools

import jax
import jax.numpy as jnp
from jax.experimental import pallas as pl
from jax.experimental.pallas import tpu as pltpu

_BN_EPS = 1e-5
_TILE_ROWS = 4096
_VMEM_LIMIT = 48 * 1024 * 1024


def _ceil_to(x, m):
    return -(-x // m) * m


def _fused_kernel(x_ref, w_ref, b_ref, o_ref, acc_ref, *,
                  batch, tile_rows, nsteps, inv_n, need_mask):
    i = pl.program_id(0)
    lin = jnp.dot(x_ref[...], w_ref[...],
                  preferred_element_type=jnp.float32) + b_ref[...]
    y = jnp.tanh(lin)
    o_ref[pl.ds(i * tile_rows, tile_rows), :] = y
    if need_mask:
        row = jax.lax.broadcasted_iota(jnp.int32, y.shape, 0) + i * tile_rows
        y = jnp.where(row < batch, y, 0.0)
    s = jnp.sum(y)
    ss = jnp.sum(y * y)
    row2 = jax.lax.broadcasted_iota(jnp.int32, acc_ref.shape, 0)
    part = jnp.where(row2 == 0, s, ss)

    @pl.when(i == 0)
    def _init():
        acc_ref[...] = part

    @pl.when(i > 0)
    def _acc():
        acc_ref[...] = acc_ref[...] + part

    @pl.when(i == nsteps - 1)
    def _normalize():
        p = acc_ref[...]
        r = jax.lax.broadcasted_iota(jnp.int32, p.shape, 0)
        total = jnp.sum(jnp.where(r == 0, p, 0.0)) * (1.0 / 128.0)
        total_sq = jnp.sum(jnp.where(r == 1, p, 0.0)) * (1.0 / 128.0)
        mean = total * inv_n
        var = jnp.maximum(total_sq * inv_n - mean * mean, 0.0)
        inv_std = jax.lax.rsqrt(var + jnp.float32(_BN_EPS))
        o_ref[...] = (o_ref[...] - mean) * inv_std


def kernel(x, weight, bias):
    B, G, F = x.shape
    GF = G * F

    x = x.astype(jnp.float32)
    weight = weight.astype(jnp.float32)
    bias = bias.astype(jnp.float32).reshape(1, G)

    TILE = min(_TILE_ROWS, _ceil_to(B, 8))
    Bp = _ceil_to(B, TILE)
    nt = Bp // TILE

    x_flat = x.reshape(B, GF)
    if Bp != B:
        x_flat = jnp.pad(x_flat, ((0, Bp - B), (0, 0)))

    w_bd = (weight[:, :, None] * jnp.eye(G, dtype=jnp.float32)[:, None, :]
            ).reshape(GF, G)

    fk = functools.partial(
        _fused_kernel, batch=B, tile_rows=TILE, nsteps=nt,
        inv_n=1.0 / float(B * G), need_mask=(Bp != B))
    out = pl.pallas_call(
        fk,
        out_shape=jax.ShapeDtypeStruct((Bp, G), jnp.float32),
        grid=(nt,),
        in_specs=[
            pl.BlockSpec((TILE, GF), lambda i: (i, 0)),
            pl.BlockSpec((GF, G), lambda i: (0, 0)),
            pl.BlockSpec((1, G), lambda i: (0, 0)),
        ],
        out_specs=pl.BlockSpec((Bp, G), lambda i: (0, 0)),
        scratch_shapes=[pltpu.VMEM((2, 128), jnp.float32)],
        compiler_params=pltpu.CompilerParams(
            dimension_semantics=("arbitrary",),
            vmem_limit_bytes=_VMEM_LIMIT,
        ),
    )(x_flat, w_bd, bias)

    return out[:B].reshape(B, G, 1)
